# baseline (speedup 1.0000x reference)
import jax
import jax.numpy as jnp
from jax.experimental import pallas as pl
from jax.experimental.pallas import tpu as pltpu

_N = 20000
_LANES = 128
_ROWS = 160
_CHUNKS = _ROWS // 8
_PAD = _ROWS * _LANES
_BIG = 2 * _PAD
_MAX_DET = 300
_IOU_THR = 0.45
_CONF_THR = 0.25


def _slane_all(v, op):
    for sh in (4, 2, 1):
        v = op(v, pltpu.roll(v, sh, axis=0))
    return v


def _nms_kernel(x1_ref, y1_ref, x2_ref, y2_ref, s_ref, out_ref,
                live_ref, area_ref, flat_ref):
    x1 = x1_ref[...]
    y1 = y1_ref[...]
    x2 = x2_ref[...]
    y2 = y2_ref[...]
    area_ref[...] = (x2 - x1) * (y2 - y1)

    row_i = jax.lax.broadcasted_iota(jnp.int32, (_ROWS, _LANES), 0)
    col_i = jax.lax.broadcasted_iota(jnp.int32, (_ROWS, _LANES), 1)
    flat_ref[...] = (_LANES - 1 - col_i) * _ROWS + row_i
    lane_i = jax.lax.broadcasted_iota(jnp.int32, (1, _LANES), 1)

    s0 = s_ref[...]
    s0 = jnp.where(s0 >= _CONF_THR, s0, 0.0)
    live_ref[...] = s0

    def _winner(get_s):
        m8 = get_s(0)
        for k in range(1, _CHUNKS):
            m8 = jnp.maximum(m8, get_s(k))
        colmax = _slane_all(m8, jnp.maximum)
        cand = jnp.full((8, _LANES), _BIG, jnp.int32)
        for k in range(_CHUNKS):
            sl = slice(8 * k, 8 * (k + 1))
            hit = get_s(k) == colmax
            cand = jnp.minimum(cand, jnp.where(hit, flat_ref[sl, :], _BIG))
        colidx = _slane_all(cand, jnp.minimum)
        cx1 = jnp.zeros((8, _LANES), jnp.float32)
        cy1 = jnp.zeros((8, _LANES), jnp.float32)
        cx2 = jnp.zeros((8, _LANES), jnp.float32)
        cy2 = jnp.zeros((8, _LANES), jnp.float32)
        for k in range(_CHUNKS):
            sl = slice(8 * k, 8 * (k + 1))
            ex = flat_ref[sl, :] == colidx
            cx1 = cx1 + jnp.where(ex, x1_ref[sl, :], 0.0)
            cy1 = cy1 + jnp.where(ex, y1_ref[sl, :], 0.0)
            cx2 = cx2 + jnp.where(ex, x2_ref[sl, :], 0.0)
            cy2 = cy2 + jnp.where(ex, y2_ref[sl, :], 0.0)
        cx1 = _slane_all(cx1, jnp.add)[0:1, :]
        cy1 = _slane_all(cy1, jnp.add)[0:1, :]
        cx2 = _slane_all(cx2, jnp.add)[0:1, :]
        cy2 = _slane_all(cy2, jnp.add)[0:1, :]
        cm = colmax[0:1, :]
        a = jnp.argmax(cm, axis=1, keepdims=True)
        hitlane = lane_i == a.astype(jnp.int32)
        wm = jnp.sum(jnp.where(hitlane, cm, 0.0), axis=1, keepdims=True)
        wx1 = jnp.sum(jnp.where(hitlane, cx1, 0.0), axis=1, keepdims=True)
        wy1 = jnp.sum(jnp.where(hitlane, cy1, 0.0), axis=1, keepdims=True)
        wx2 = jnp.sum(jnp.where(hitlane, cx2, 0.0), axis=1, keepdims=True)
        wy2 = jnp.sum(jnp.where(hitlane, cy2, 0.0), axis=1, keepdims=True)
        return (wm + jnp.zeros((1, _LANES), jnp.float32),
                wx1 + jnp.zeros((1, _LANES), jnp.float32),
                wy1 + jnp.zeros((1, _LANES), jnp.float32),
                wx2 + jnp.zeros((1, _LANES), jnp.float32),
                wy2 + jnp.zeros((1, _LANES), jnp.float32))

    w0 = _winner(lambda k: live_ref[8 * k:8 * (k + 1), :])

    def body(i, w):
        m, bx1, by1, bx2, by2 = w

        entry = (jnp.where(lane_i == 0, bx1, 0.0)
                 + jnp.where(lane_i == 1, by1, 0.0)
                 + jnp.where(lane_i == 2, bx2, 0.0)
                 + jnp.where(lane_i == 3, by2, 0.0)
                 + jnp.where(lane_i == 4, m, 0.0))
        out_ref[pl.ds(i, 1), :] = jnp.where(m > 0.0, entry, 0.0)

        barea = (bx2 - bx1) * (by2 - by1)
        s_chunks = []
        for k in range(_CHUNKS):
            sl = slice(8 * k, 8 * (k + 1))
            xx1 = jnp.maximum(bx1, x1_ref[sl, :])
            yy1 = jnp.maximum(by1, y1_ref[sl, :])
            xx2 = jnp.minimum(bx2, x2_ref[sl, :])
            yy2 = jnp.minimum(by2, y2_ref[sl, :])
            inter = (jnp.maximum(xx2 - xx1, 0.0)
                     * jnp.maximum(yy2 - yy1, 0.0))
            iou = inter / (barea + area_ref[sl, :] - inter + 1e-9)
            sc = jnp.where(iou > _IOU_THR, 0.0, live_ref[sl, :])
            live_ref[sl, :] = sc
            s_chunks.append(sc)

        return _winner(lambda k: s_chunks[k])

    jax.lax.fori_loop(0, _MAX_DET, body, w0, unroll=False)


def kernel(boxes, scores):
    pb = jnp.pad(boxes, ((0, _PAD - _N), (0, 0)))
    x1 = pb[:, 0].reshape(_LANES, _ROWS)[::-1].T
    y1 = pb[:, 1].reshape(_LANES, _ROWS)[::-1].T
    x2 = pb[:, 2].reshape(_LANES, _ROWS)[::-1].T
    y2 = pb[:, 3].reshape(_LANES, _ROWS)[::-1].T
    s = jnp.pad(scores, (0, _PAD - _N)).reshape(_LANES, _ROWS)[::-1].T

    out = pl.pallas_call(
        _nms_kernel,
        out_shape=jax.ShapeDtypeStruct((_MAX_DET, _LANES), jnp.float32),
        scratch_shapes=[pltpu.VMEM((_ROWS, _LANES), jnp.float32),
                        pltpu.VMEM((_ROWS, _LANES), jnp.float32),
                        pltpu.VMEM((_ROWS, _LANES), jnp.int32)],
    )(x1, y1, x2, y2, s)
    return out[:, :5]


# speedup vs baseline: 1.1249x; 1.1249x over previous
import jax
import jax.numpy as jnp
from jax.experimental import pallas as pl
from jax.experimental.pallas import tpu as pltpu

_N = 20000
_LANES = 128
_ROWS = 160
_CHUNKS = _ROWS // 8
_PAD = _ROWS * _LANES
_BIG = 2 * _PAD
_MAX_DET = 300
_IOU_THR = 0.45
_CONF_THR = 0.25


def _slane_all(v, op):
    for sh in (4, 2, 1):
        v = op(v, pltpu.roll(v, sh, axis=0))
    return v


def _nms_kernel(x1_ref, y1_ref, x2_ref, y2_ref, s_ref, out_ref,
                live_ref, area_ref, flat_ref):
    x1 = x1_ref[...]
    y1 = y1_ref[...]
    x2 = x2_ref[...]
    y2 = y2_ref[...]
    area_ref[...] = (x2 - x1) * (y2 - y1)

    row_i = jax.lax.broadcasted_iota(jnp.int32, (_ROWS, _LANES), 0)
    col_i = jax.lax.broadcasted_iota(jnp.int32, (_ROWS, _LANES), 1)
    flat_ref[...] = (_LANES - 1 - col_i) * _ROWS + row_i
    lane_i = jax.lax.broadcasted_iota(jnp.int32, (1, _LANES), 1)

    s0 = s_ref[...]
    s0 = jnp.where(s0 >= _CONF_THR, s0, 0.0)
    live_ref[...] = s0

    def _lane_bcast_sum(hit, v):
        return jnp.sum(jnp.where(hit, v, 0.0), axis=1, keepdims=True) \
            + jnp.zeros((1, _LANES), jnp.float32)

    def _iou_1x(m_, ax1, ay1, ax2, ay2, bx1, by1, bx2, by2):
        aarea = (ax2 - ax1) * (ay2 - ay1)
        barea = (bx2 - bx1) * (by2 - by1)
        xx1 = jnp.maximum(ax1, bx1)
        yy1 = jnp.maximum(ay1, by1)
        xx2 = jnp.minimum(ax2, bx2)
        yy2 = jnp.minimum(ay2, by2)
        inter = jnp.maximum(xx2 - xx1, 0.0) * jnp.maximum(yy2 - yy1, 0.0)
        return inter / (aarea + barea - inter + 1e-9)

    def _top2(get_s):
        m8 = get_s(0)
        for k in range(1, _CHUNKS):
            m8 = jnp.maximum(m8, get_s(k))
        colmax = _slane_all(m8, jnp.maximum)
        cand = jnp.full((8, _LANES), _BIG, jnp.int32)
        for k in range(_CHUNKS):
            sl = slice(8 * k, 8 * (k + 1))
            hit = get_s(k) == colmax
            cand = jnp.minimum(cand, jnp.where(hit, flat_ref[sl, :], _BIG))
        colidx = _slane_all(cand, jnp.minimum)
        m8b = jnp.zeros((8, _LANES), jnp.float32)
        for k in range(_CHUNKS):
            sl = slice(8 * k, 8 * (k + 1))
            ex = flat_ref[sl, :] == colidx
            m8b = jnp.maximum(m8b, jnp.where(ex, 0.0, get_s(k)))
        colmax2 = _slane_all(m8b, jnp.maximum)
        cand2 = jnp.full((8, _LANES), _BIG, jnp.int32)
        for k in range(_CHUNKS):
            sl = slice(8 * k, 8 * (k + 1))
            fl = flat_ref[sl, :]
            hit2 = (get_s(k) == colmax2) & (fl != colidx)
            cand2 = jnp.minimum(cand2, jnp.where(hit2, fl, _BIG))
        colidx2 = _slane_all(cand2, jnp.minimum)
        c1 = [jnp.zeros((8, _LANES), jnp.float32) for _ in range(4)]
        c2 = [jnp.zeros((8, _LANES), jnp.float32) for _ in range(4)]
        for k in range(_CHUNKS):
            sl = slice(8 * k, 8 * (k + 1))
            fl = flat_ref[sl, :]
            ex1 = fl == colidx
            ex2 = fl == colidx2
            c1[0] = c1[0] + jnp.where(ex1, x1_ref[sl, :], 0.0)
            c1[1] = c1[1] + jnp.where(ex1, y1_ref[sl, :], 0.0)
            c1[2] = c1[2] + jnp.where(ex1, x2_ref[sl, :], 0.0)
            c1[3] = c1[3] + jnp.where(ex1, y2_ref[sl, :], 0.0)
            c2[0] = c2[0] + jnp.where(ex2, x1_ref[sl, :], 0.0)
            c2[1] = c2[1] + jnp.where(ex2, y1_ref[sl, :], 0.0)
            c2[2] = c2[2] + jnp.where(ex2, x2_ref[sl, :], 0.0)
            c2[3] = c2[3] + jnp.where(ex2, y2_ref[sl, :], 0.0)
        c1 = [_slane_all(c, jnp.add)[0:1, :] for c in c1]
        c2 = [_slane_all(c, jnp.add)[0:1, :] for c in c2]
        cm1 = colmax[0:1, :]
        cm2 = colmax2[0:1, :]

        a1 = jnp.argmax(cm1, axis=1, keepdims=True).astype(jnp.int32)
        hit1 = lane_i == a1
        mm = jnp.where(hit1, cm2, cm1)
        mx1 = jnp.where(hit1, c2[0], c1[0])
        my1 = jnp.where(hit1, c2[1], c1[1])
        mx2 = jnp.where(hit1, c2[2], c1[2])
        my2 = jnp.where(hit1, c2[3], c1[3])
        wm1 = _lane_bcast_sum(hit1, cm1)
        wx1 = _lane_bcast_sum(hit1, c1[0])
        wy1 = _lane_bcast_sum(hit1, c1[1])
        wx2 = _lane_bcast_sum(hit1, c1[2])
        wy2 = _lane_bcast_sum(hit1, c1[3])
        a2 = jnp.argmax(mm, axis=1, keepdims=True).astype(jnp.int32)
        hit2 = lane_i == a2
        iou_all = _iou_1x(None, wx1, wy1, wx2, wy2, mx1, my1, mx2, my2)
        okv = jnp.where(iou_all > _IOU_THR, 0.0, 1.0)
        wm2 = _lane_bcast_sum(hit2, mm)
        vx1 = _lane_bcast_sum(hit2, mx1)
        vy1 = _lane_bcast_sum(hit2, my1)
        vx2 = _lane_bcast_sum(hit2, mx2)
        vy2 = _lane_bcast_sum(hit2, my2)
        acc = _lane_bcast_sum(hit2, okv)
        return ((wm1, wx1, wy1, wx2, wy2),
                (wm2, vx1, vy1, vx2, vy2), acc)

    w1_0, w2_0, acc_0 = _top2(lambda k: live_ref[8 * k:8 * (k + 1), :])

    def _entry(w):
        m, bx1, by1, bx2, by2 = w
        e = (jnp.where(lane_i == 0, bx1, 0.0)
             + jnp.where(lane_i == 1, by1, 0.0)
             + jnp.where(lane_i == 2, bx2, 0.0)
             + jnp.where(lane_i == 3, by2, 0.0)
             + jnp.where(lane_i == 4, m, 0.0))
        return jnp.where(m > 0.0, e, 0.0)

    def cond(state):
        return state[0] < _MAX_DET

    def body(state):
        k, w1, w2, acc = state
        m1, bx1, by1, bx2, by2 = w1
        m2, ex1, ey1, ex2, ey2 = w2
        acc_s = acc[0, 0] > 0.5

        out_ref[pl.ds(k, 1), :] = _entry(w1)

        @pl.when(acc_s & (k < _MAX_DET - 1))
        def _():
            out_ref[pl.ds(k + 1, 1), :] = _entry(w2)

        k_new = k + 1 + acc_s.astype(jnp.int32)

        barea1 = (bx2 - bx1) * (by2 - by1)
        barea2 = (ex2 - ex1) * (ey2 - ey1)
        accv = acc > 0.5
        s_chunks = []
        for c in range(_CHUNKS):
            sl = slice(8 * c, 8 * (c + 1))
            cx1 = x1_ref[sl, :]
            cy1 = y1_ref[sl, :]
            cx2 = x2_ref[sl, :]
            cy2 = y2_ref[sl, :]
            ar = area_ref[sl, :]
            i1 = (jnp.maximum(jnp.minimum(bx2, cx2) - jnp.maximum(bx1, cx1), 0.0)
                  * jnp.maximum(jnp.minimum(by2, cy2) - jnp.maximum(by1, cy1), 0.0))
            iou1 = i1 / (barea1 + ar - i1 + 1e-9)
            i2 = (jnp.maximum(jnp.minimum(ex2, cx2) - jnp.maximum(ex1, cx1), 0.0)
                  * jnp.maximum(jnp.minimum(ey2, cy2) - jnp.maximum(ey1, cy1), 0.0))
            iou2 = i2 / (barea2 + ar - i2 + 1e-9)
            supp = (iou1 > _IOU_THR) | ((iou2 > _IOU_THR) & accv)
            sc = jnp.where(supp, 0.0, live_ref[sl, :])
            live_ref[sl, :] = sc
            s_chunks.append(sc)

        w1n, w2n, accn = _top2(lambda c: s_chunks[c])
        return (k_new, w1n, w2n, accn)

    jax.lax.while_loop(cond, body, (jnp.int32(0), w1_0, w2_0, acc_0))


def kernel(boxes, scores):
    pb = jnp.pad(boxes, ((0, _PAD - _N), (0, 0)))
    x1 = pb[:, 0].reshape(_LANES, _ROWS)[::-1].T
    y1 = pb[:, 1].reshape(_LANES, _ROWS)[::-1].T
    x2 = pb[:, 2].reshape(_LANES, _ROWS)[::-1].T
    y2 = pb[:, 3].reshape(_LANES, _ROWS)[::-1].T
    s = jnp.pad(scores, (0, _PAD - _N)).reshape(_LANES, _ROWS)[::-1].T

    out = pl.pallas_call(
        _nms_kernel,
        out_shape=jax.ShapeDtypeStruct((_MAX_DET, _LANES), jnp.float32),
        scratch_shapes=[pltpu.VMEM((_ROWS, _LANES), jnp.float32),
                        pltpu.VMEM((_ROWS, _LANES), jnp.float32),
                        pltpu.VMEM((_ROWS, _LANES), jnp.int32)],
    )(x1, y1, x2, y2, s)
    return out[:, :5]


# speedup vs baseline: 1.1282x; 1.0029x over previous
import jax
import jax.numpy as jnp
from jax.experimental import pallas as pl
from jax.experimental.pallas import tpu as pltpu

_N = 20000
_LANES = 128
_ROWS = 160
_CHUNKS = _ROWS // 8
_PAD = _ROWS * _LANES
_BIG = 2 * _PAD
_MAX_DET = 300
_IOU_THR = 0.45
_CONF_THR = 0.25


def _slane_all(v, op):
    for sh in (4, 2, 1):
        v = op(v, pltpu.roll(v, sh, axis=0))
    return v


def _nms_kernel(x1_ref, y1_ref, x2_ref, y2_ref, s_ref, out_ref,
                live_ref, area_ref, flat_ref):
    x1 = x1_ref[...]
    y1 = y1_ref[...]
    x2 = x2_ref[...]
    y2 = y2_ref[...]
    area_ref[...] = (x2 - x1) * (y2 - y1)

    row_i = jax.lax.broadcasted_iota(jnp.int32, (_ROWS, _LANES), 0)
    col_i = jax.lax.broadcasted_iota(jnp.int32, (_ROWS, _LANES), 1)
    flat_ref[...] = (_LANES - 1 - col_i) * _ROWS + row_i
    lane_i = jax.lax.broadcasted_iota(jnp.int32, (1, _LANES), 1)

    s0 = s_ref[...]
    s0 = jnp.where(s0 >= _CONF_THR, s0, 0.0)
    live_ref[...] = s0

    def _lane_bcast_sum(hit, v):
        return jnp.sum(jnp.where(hit, v, 0.0), axis=1, keepdims=True) \
            + jnp.zeros((1, _LANES), jnp.float32)

    def _iou_1x(m_, ax1, ay1, ax2, ay2, bx1, by1, bx2, by2):
        aarea = (ax2 - ax1) * (ay2 - ay1)
        barea = (bx2 - bx1) * (by2 - by1)
        xx1 = jnp.maximum(ax1, bx1)
        yy1 = jnp.maximum(ay1, by1)
        xx2 = jnp.minimum(ax2, bx2)
        yy2 = jnp.minimum(ay2, by2)
        inter = jnp.maximum(xx2 - xx1, 0.0) * jnp.maximum(yy2 - yy1, 0.0)
        return inter / (aarea + barea - inter + 1e-9)

    def _top2(get_s):
        m8 = get_s(0)
        for k in range(1, _CHUNKS):
            m8 = jnp.maximum(m8, get_s(k))
        colmax = _slane_all(m8, jnp.maximum)
        cand = jnp.full((8, _LANES), _BIG, jnp.int32)
        for k in range(_CHUNKS):
            sl = slice(8 * k, 8 * (k + 1))
            hit = get_s(k) == colmax
            cand = jnp.minimum(cand, jnp.where(hit, flat_ref[sl, :], _BIG))
        colidx = _slane_all(cand, jnp.minimum)
        m8b = jnp.zeros((8, _LANES), jnp.float32)
        for k in range(_CHUNKS):
            sl = slice(8 * k, 8 * (k + 1))
            ex = flat_ref[sl, :] == colidx
            m8b = jnp.maximum(m8b, jnp.where(ex, 0.0, get_s(k)))
        colmax2 = _slane_all(m8b, jnp.maximum)
        cand2 = jnp.full((8, _LANES), _BIG, jnp.int32)
        for k in range(_CHUNKS):
            sl = slice(8 * k, 8 * (k + 1))
            fl = flat_ref[sl, :]
            hit2 = (get_s(k) == colmax2) & (fl != colidx)
            cand2 = jnp.minimum(cand2, jnp.where(hit2, fl, _BIG))
        colidx2 = _slane_all(cand2, jnp.minimum)
        c1 = [jnp.zeros((8, _LANES), jnp.float32) for _ in range(4)]
        c2 = [jnp.zeros((8, _LANES), jnp.float32) for _ in range(4)]
        for k in range(_CHUNKS):
            sl = slice(8 * k, 8 * (k + 1))
            fl = flat_ref[sl, :]
            ex1 = fl == colidx
            ex2 = fl == colidx2
            c1[0] = c1[0] + jnp.where(ex1, x1_ref[sl, :], 0.0)
            c1[1] = c1[1] + jnp.where(ex1, y1_ref[sl, :], 0.0)
            c1[2] = c1[2] + jnp.where(ex1, x2_ref[sl, :], 0.0)
            c1[3] = c1[3] + jnp.where(ex1, y2_ref[sl, :], 0.0)
            c2[0] = c2[0] + jnp.where(ex2, x1_ref[sl, :], 0.0)
            c2[1] = c2[1] + jnp.where(ex2, y1_ref[sl, :], 0.0)
            c2[2] = c2[2] + jnp.where(ex2, x2_ref[sl, :], 0.0)
            c2[3] = c2[3] + jnp.where(ex2, y2_ref[sl, :], 0.0)
        c1 = [_slane_all(c, jnp.add)[0:1, :] for c in c1]
        c2 = [_slane_all(c, jnp.add)[0:1, :] for c in c2]
        cm1 = colmax[0:1, :]
        cm2 = colmax2[0:1, :]

        a1 = jnp.argmax(cm1, axis=1, keepdims=True).astype(jnp.int32)
        hit1 = lane_i == a1
        mm = jnp.where(hit1, cm2, cm1)
        mx1 = jnp.where(hit1, c2[0], c1[0])
        my1 = jnp.where(hit1, c2[1], c1[1])
        mx2 = jnp.where(hit1, c2[2], c1[2])
        my2 = jnp.where(hit1, c2[3], c1[3])
        wm1 = _lane_bcast_sum(hit1, cm1)
        wx1 = _lane_bcast_sum(hit1, c1[0])
        wy1 = _lane_bcast_sum(hit1, c1[1])
        wx2 = _lane_bcast_sum(hit1, c1[2])
        wy2 = _lane_bcast_sum(hit1, c1[3])
        a2 = jnp.argmax(mm, axis=1, keepdims=True).astype(jnp.int32)
        hit2 = lane_i == a2
        iou_all = _iou_1x(None, wx1, wy1, wx2, wy2, mx1, my1, mx2, my2)
        okv = jnp.where(iou_all > _IOU_THR, 0.0, 1.0)
        wm2 = _lane_bcast_sum(hit2, mm)
        vx1 = _lane_bcast_sum(hit2, mx1)
        vy1 = _lane_bcast_sum(hit2, my1)
        vx2 = _lane_bcast_sum(hit2, mx2)
        vy2 = _lane_bcast_sum(hit2, my2)
        acc = _lane_bcast_sum(hit2, okv)
        return ((wm1, wx1, wy1, wx2, wy2),
                (wm2, vx1, vy1, vx2, vy2), acc)

    w1_0, w2_0, acc_0 = _top2(lambda k: live_ref[8 * k:8 * (k + 1), :])

    def _entry(w):
        m, bx1, by1, bx2, by2 = w
        e = (jnp.where(lane_i == 0, bx1, 0.0)
             + jnp.where(lane_i == 1, by1, 0.0)
             + jnp.where(lane_i == 2, bx2, 0.0)
             + jnp.where(lane_i == 3, by2, 0.0)
             + jnp.where(lane_i == 4, m, 0.0))
        return jnp.where(m > 0.0, e, 0.0)

    def cond(state):
        return state[0] < _MAX_DET

    def body(state):
        k, w1, w2, acc = state
        m1, bx1, by1, bx2, by2 = w1
        m2, ex1, ey1, ex2, ey2 = w2
        acc_s = acc[0, 0] > 0.5

        out_ref[pl.ds(k, 1), :] = _entry(w1)

        @pl.when(acc_s & (k < _MAX_DET - 1))
        def _():
            out_ref[pl.ds(k + 1, 1), :] = _entry(w2)

        k_new = k + 1 + acc_s.astype(jnp.int32)

        barea1 = (bx2 - bx1) * (by2 - by1)
        barea2 = (ex2 - ex1) * (ey2 - ey1)
        accv = acc > 0.5
        ex1 = jnp.where(accv, ex1, 0.0)
        ey1 = jnp.where(accv, ey1, 0.0)
        ex2 = jnp.where(accv, ex2, 0.0)
        ey2 = jnp.where(accv, ey2, 0.0)
        barea2 = jnp.where(accv, barea2, 0.0)
        for c in range(_CHUNKS):
            sl = slice(8 * c, 8 * (c + 1))
            cx1 = x1_ref[sl, :]
            cy1 = y1_ref[sl, :]
            cx2 = x2_ref[sl, :]
            cy2 = y2_ref[sl, :]
            ar = area_ref[sl, :]
            i1 = (jnp.maximum(jnp.minimum(bx2, cx2) - jnp.maximum(bx1, cx1), 0.0)
                  * jnp.maximum(jnp.minimum(by2, cy2) - jnp.maximum(by1, cy1), 0.0))
            iou1 = i1 / (barea1 + ar - i1 + 1e-9)
            i2 = (jnp.maximum(jnp.minimum(ex2, cx2) - jnp.maximum(ex1, cx1), 0.0)
                  * jnp.maximum(jnp.minimum(ey2, cy2) - jnp.maximum(ey1, cy1), 0.0))
            iou2 = i2 / (barea2 + ar - i2 + 1e-9)
            supp = (iou1 > _IOU_THR) | (iou2 > _IOU_THR)
            live_ref[sl, :] = jnp.where(supp, 0.0, live_ref[sl, :])

        w1n, w2n, accn = _top2(lambda c: live_ref[8 * c:8 * (c + 1), :])
        return (k_new, w1n, w2n, accn)

    jax.lax.while_loop(cond, body, (jnp.int32(0), w1_0, w2_0, acc_0))


def kernel(boxes, scores):
    pb = jnp.pad(boxes, ((0, _PAD - _N), (0, 0)))
    x1 = pb[:, 0].reshape(_LANES, _ROWS)[::-1].T
    y1 = pb[:, 1].reshape(_LANES, _ROWS)[::-1].T
    x2 = pb[:, 2].reshape(_LANES, _ROWS)[::-1].T
    y2 = pb[:, 3].reshape(_LANES, _ROWS)[::-1].T
    s = jnp.pad(scores, (0, _PAD - _N)).reshape(_LANES, _ROWS)[::-1].T

    out = pl.pallas_call(
        _nms_kernel,
        out_shape=jax.ShapeDtypeStruct((_MAX_DET, _LANES), jnp.float32),
        scratch_shapes=[pltpu.VMEM((_ROWS, _LANES), jnp.float32),
                        pltpu.VMEM((_ROWS, _LANES), jnp.float32),
                        pltpu.VMEM((_ROWS, _LANES), jnp.int32)],
    )(x1, y1, x2, y2, s)
    return out[:, :5]


# speedup vs baseline: 1.2316x; 1.0917x over previous
import jax
import jax.numpy as jnp
from jax.experimental import pallas as pl
from jax.experimental.pallas import tpu as pltpu

_N = 20000
_LANES = 128
_ROWS = 160
_CHUNKS = _ROWS // 8
_PAD = _ROWS * _LANES
_BIG = 2 * _PAD
_MAX_DET = 300
_IOU_THR = 0.45
_CONF_THR = 0.25


def _slane_all(v, op):
    for sh in (4, 2, 1):
        v = op(v, pltpu.roll(v, sh, axis=0))
    return v


def _nms_kernel(x1_ref, y1_ref, x2_ref, y2_ref, s_ref, out_ref,
                live_ref, area_ref, flat_ref):
    x1 = x1_ref[...]
    y1 = y1_ref[...]
    x2 = x2_ref[...]
    y2 = y2_ref[...]
    area_ref[...] = (x2 - x1) * (y2 - y1)

    row_i = jax.lax.broadcasted_iota(jnp.int32, (_ROWS, _LANES), 0)
    col_i = jax.lax.broadcasted_iota(jnp.int32, (_ROWS, _LANES), 1)
    flat_ref[...] = (_LANES - 1 - col_i) * _ROWS + row_i
    lane_i = jax.lax.broadcasted_iota(jnp.int32, (1, _LANES), 1)

    s0 = s_ref[...]
    s0 = jnp.where(s0 >= _CONF_THR, s0, 0.0)
    live_ref[...] = s0

    def _lane_bcast_sum(hit, v):
        return jnp.sum(jnp.where(hit, v, 0.0), axis=1, keepdims=True) \
            + jnp.zeros((1, _LANES), jnp.float32)

    def _iou_1x(m_, ax1, ay1, ax2, ay2, bx1, by1, bx2, by2):
        aarea = (ax2 - ax1) * (ay2 - ay1)
        barea = (bx2 - bx1) * (by2 - by1)
        xx1 = jnp.maximum(ax1, bx1)
        yy1 = jnp.maximum(ay1, by1)
        xx2 = jnp.minimum(ax2, bx2)
        yy2 = jnp.minimum(ay2, by2)
        inter = jnp.maximum(xx2 - xx1, 0.0) * jnp.maximum(yy2 - yy1, 0.0)
        return inter / (aarea + barea - inter + 1e-9)

    def _top2(get_s):
        m8 = get_s(0)
        for k in range(1, _CHUNKS):
            m8 = jnp.maximum(m8, get_s(k))
        colmax = _slane_all(m8, jnp.maximum)
        cand = jnp.full((8, _LANES), _BIG, jnp.int32)
        for k in range(_CHUNKS):
            sl = slice(8 * k, 8 * (k + 1))
            hit = get_s(k) == colmax
            cand = jnp.minimum(cand, jnp.where(hit, flat_ref[sl, :], _BIG))
        colidx = _slane_all(cand, jnp.minimum)
        m8b = jnp.zeros((8, _LANES), jnp.float32)
        for k in range(_CHUNKS):
            sl = slice(8 * k, 8 * (k + 1))
            ex = flat_ref[sl, :] == colidx
            m8b = jnp.maximum(m8b, jnp.where(ex, 0.0, get_s(k)))
        colmax2 = _slane_all(m8b, jnp.maximum)
        cand2 = jnp.full((8, _LANES), _BIG, jnp.int32)
        for k in range(_CHUNKS):
            sl = slice(8 * k, 8 * (k + 1))
            fl = flat_ref[sl, :]
            hit2 = (get_s(k) == colmax2) & (fl != colidx)
            cand2 = jnp.minimum(cand2, jnp.where(hit2, fl, _BIG))
        colidx2 = _slane_all(cand2, jnp.minimum)
        c1 = [jnp.zeros((8, _LANES), jnp.float32) for _ in range(4)]
        c2 = [jnp.zeros((8, _LANES), jnp.float32) for _ in range(4)]
        for k in range(_CHUNKS):
            sl = slice(8 * k, 8 * (k + 1))
            fl = flat_ref[sl, :]
            ex1 = fl == colidx
            ex2 = fl == colidx2
            c1[0] = c1[0] + jnp.where(ex1, x1_ref[sl, :], 0.0)
            c1[1] = c1[1] + jnp.where(ex1, y1_ref[sl, :], 0.0)
            c1[2] = c1[2] + jnp.where(ex1, x2_ref[sl, :], 0.0)
            c1[3] = c1[3] + jnp.where(ex1, y2_ref[sl, :], 0.0)
            c2[0] = c2[0] + jnp.where(ex2, x1_ref[sl, :], 0.0)
            c2[1] = c2[1] + jnp.where(ex2, y1_ref[sl, :], 0.0)
            c2[2] = c2[2] + jnp.where(ex2, x2_ref[sl, :], 0.0)
            c2[3] = c2[3] + jnp.where(ex2, y2_ref[sl, :], 0.0)
        c1 = [_slane_all(c, jnp.add)[0:1, :] for c in c1]
        c2 = [_slane_all(c, jnp.add)[0:1, :] for c in c2]
        cm1 = colmax[0:1, :]
        cm2 = colmax2[0:1, :]

        a1 = jnp.argmax(cm1, axis=1, keepdims=True).astype(jnp.int32)
        hit1 = lane_i == a1
        mm = jnp.where(hit1, cm2, cm1)
        mx1 = jnp.where(hit1, c2[0], c1[0])
        my1 = jnp.where(hit1, c2[1], c1[1])
        mx2 = jnp.where(hit1, c2[2], c1[2])
        my2 = jnp.where(hit1, c2[3], c1[3])
        wm1 = _lane_bcast_sum(hit1, cm1)
        wx1 = _lane_bcast_sum(hit1, c1[0])
        wy1 = _lane_bcast_sum(hit1, c1[1])
        wx2 = _lane_bcast_sum(hit1, c1[2])
        wy2 = _lane_bcast_sum(hit1, c1[3])
        a2 = jnp.argmax(mm, axis=1, keepdims=True).astype(jnp.int32)
        hit2 = lane_i == a2
        iou_all = _iou_1x(None, wx1, wy1, wx2, wy2, mx1, my1, mx2, my2)
        okv = jnp.where(iou_all > _IOU_THR, 0.0, 1.0)
        wm2 = _lane_bcast_sum(hit2, mm)
        vx1 = _lane_bcast_sum(hit2, mx1)
        vy1 = _lane_bcast_sum(hit2, my1)
        vx2 = _lane_bcast_sum(hit2, mx2)
        vy2 = _lane_bcast_sum(hit2, my2)
        acc = _lane_bcast_sum(hit2, okv)
        return ((wm1, wx1, wy1, wx2, wy2),
                (wm2, vx1, vy1, vx2, vy2), acc)

    w1_0, w2_0, acc_0 = _top2(lambda k: live_ref[8 * k:8 * (k + 1), :])

    def _entry(w):
        m, bx1, by1, bx2, by2 = w
        e = (jnp.where(lane_i == 0, bx1, 0.0)
             + jnp.where(lane_i == 1, by1, 0.0)
             + jnp.where(lane_i == 2, bx2, 0.0)
             + jnp.where(lane_i == 3, by2, 0.0)
             + jnp.where(lane_i == 4, m, 0.0))
        return jnp.where(m > 0.0, e, 0.0)

    def _suppress(bx1, by1, bx2, by2, barea):
        for c in range(_CHUNKS):
            sl = slice(8 * c, 8 * (c + 1))
            inter = (jnp.maximum(jnp.minimum(bx2, x2_ref[sl, :])
                                 - jnp.maximum(bx1, x1_ref[sl, :]), 0.0)
                     * jnp.maximum(jnp.minimum(by2, y2_ref[sl, :])
                                   - jnp.maximum(by1, y1_ref[sl, :]), 0.0))
            iou = inter / (barea + area_ref[sl, :] - inter + 1e-9)
            live_ref[sl, :] = jnp.where(iou > _IOU_THR, 0.0, live_ref[sl, :])

    _m1_0, _b1, _b2, _b3, _b4 = w1_0
    _suppress(_b1, _b2, _b3, _b4, (_b3 - _b1) * (_b4 - _b2))

    def cond(state):
        return state[0] < _MAX_DET

    def body(state):
        k, w1, w2, acc = state
        m2, ex1, ey1, ex2, ey2 = w2
        acc_s = acc[0, 0] > 0.5

        out_ref[pl.ds(k, 1), :] = _entry(w1)

        @pl.when(acc_s & (k < _MAX_DET - 1))
        def _():
            out_ref[pl.ds(k + 1, 1), :] = _entry(w2)

        k_new = k + 1 + acc_s.astype(jnp.int32)

        barea2 = (ex2 - ex1) * (ey2 - ey1)
        accv = acc > 0.5
        ex1 = jnp.where(accv, ex1, 0.0)
        ey1 = jnp.where(accv, ey1, 0.0)
        ex2 = jnp.where(accv, ex2, 0.0)
        ey2 = jnp.where(accv, ey2, 0.0)
        barea2 = jnp.where(accv, barea2, 0.0)
        _suppress(ex1, ey1, ex2, ey2, barea2)

        w1n, w2n, accn = _top2(lambda c: live_ref[8 * c:8 * (c + 1), :])
        _m1n, nx1, ny1, nx2, ny2 = w1n
        _suppress(nx1, ny1, nx2, ny2, (nx2 - nx1) * (ny2 - ny1))
        return (k_new, w1n, w2n, accn)

    jax.lax.while_loop(cond, body, (jnp.int32(0), w1_0, w2_0, acc_0))


def kernel(boxes, scores):
    pb = jnp.pad(boxes, ((0, _PAD - _N), (0, 0)))
    x1 = pb[:, 0].reshape(_LANES, _ROWS)[::-1].T
    y1 = pb[:, 1].reshape(_LANES, _ROWS)[::-1].T
    x2 = pb[:, 2].reshape(_LANES, _ROWS)[::-1].T
    y2 = pb[:, 3].reshape(_LANES, _ROWS)[::-1].T
    s = jnp.pad(scores, (0, _PAD - _N)).reshape(_LANES, _ROWS)[::-1].T

    out = pl.pallas_call(
        _nms_kernel,
        out_shape=jax.ShapeDtypeStruct((_MAX_DET, _LANES), jnp.float32),
        scratch_shapes=[pltpu.VMEM((_ROWS, _LANES), jnp.float32),
                        pltpu.VMEM((_ROWS, _LANES), jnp.float32),
                        pltpu.VMEM((_ROWS, _LANES), jnp.int32)],
    )(x1, y1, x2, y2, s)
    return out[:, :5]
